# baseline (speedup 1.0000x reference)
import math

import jax
import jax.numpy as jnp
import numpy as np
from jax.experimental import pallas as pl
from jax.experimental.pallas import tpu as pltpu
from jax.sharding import Mesh, PartitionSpec as P

_NB = 32
_MD = 128
_BR = 512
_S = 2048


def _bias_block(r, table_ref):
    row = jax.lax.broadcasted_iota(jnp.int32, (_BR, _S), 0) + r * _BR
    col = jax.lax.broadcasted_iota(jnp.int32, (_BR, _S), 1)
    d = col - row
    rb = jnp.where(d > 0, _NB // 2, 0)
    a = jnp.abs(d)
    af = a.astype(jnp.float32)
    rp_if_large = _MD + jnp.log(af / _MD) / math.log(_MD / _NB) * (_NB - _MD)
    rp_if_large = jnp.minimum(rp_if_large, _MD - 1)
    large = rb.astype(jnp.float32) + rp_if_large
    small = (a + rb).astype(jnp.float32)
    out = jnp.where(a < _MD, small, large)
    bucket = jnp.clip(out, 0, _NB - 1).astype(jnp.int32)
    acc = jnp.zeros((_BR, _S), jnp.float32)
    for k in range(_NB):
        acc = jnp.where(bucket == k, table_ref[k, 0], acc)
    return acc


def _add_bias_kernel(x_ref, table_ref, o_ref, bias_ref):
    h = pl.program_id(1)

    @pl.when(h == 0)
    def _():
        bias_ref[...] = _bias_block(pl.program_id(0), table_ref)

    o_ref[...] = x_ref[...] + bias_ref[...]


def _run(x, bias_table):
    bh, s, _ = x.shape
    grid = (s // _BR, bh)
    return pl.pallas_call(
        _add_bias_kernel,
        grid=grid,
        in_specs=[
            pl.BlockSpec((1, _BR, s), lambda r, hh: (hh, r, 0)),
            pl.BlockSpec((_NB, 1), lambda r, hh: (0, 0)),
        ],
        out_specs=pl.BlockSpec((1, _BR, s), lambda r, hh: (hh, r, 0)),
        out_shape=jax.ShapeDtypeStruct((bh, s, s), jnp.float32),
        scratch_shapes=[pltpu.VMEM((_BR, s), jnp.float32)],
        compiler_params=pltpu.CompilerParams(
            dimension_semantics=("parallel", "arbitrary")
        ),
    )(x, bias_table)


def kernel(attention_scores, bias_table):
    b, h, s, _ = attention_scores.shape
    x = attention_scores.reshape(b * h, s, s)
    devs = jax.devices()
    nd = max(d for d in (1, 2, 4, 8, 16) if d <= len(devs) and (b * h) % d == 0)
    if nd > 1:
        mesh = Mesh(np.array(devs[:nd]), ("d",))
        run = jax.shard_map(
            _run,
            mesh=mesh,
            in_specs=(P("d", None, None), P(None, None)),
            out_specs=P("d", None, None),
            check_vma=False,
        )
        out = run(x, bias_table)
    else:
        out = _run(x, bias_table)
    return out.reshape(b, h, s, s)


# speedup vs baseline: 2.7451x; 2.7451x over previous
import math

import jax
import jax.numpy as jnp
import numpy as np
from jax.experimental import pallas as pl
from jax.experimental.pallas import tpu as pltpu
from jax.sharding import Mesh, PartitionSpec as P

_NB = 32
_MD = 128
_BR = 256
_S = 2048


def _bias_block(r, table_ref):
    row = jax.lax.broadcasted_iota(jnp.int32, (_BR, _S), 0) + r * _BR
    col = jax.lax.broadcasted_iota(jnp.int32, (_BR, _S), 1)
    d = col - row
    rb = jnp.where(d > 0, _NB // 2, 0)
    a = jnp.abs(d)
    af = a.astype(jnp.float32)
    rp_if_large = _MD + jnp.log(af / _MD) / math.log(_MD / _NB) * (_NB - _MD)
    rp_if_large = jnp.minimum(rp_if_large, _MD - 1)
    large = rb.astype(jnp.float32) + rp_if_large
    small = (a + rb).astype(jnp.float32)
    out = jnp.where(a < _MD, small, large)
    bucket = jnp.clip(out, 0, _NB - 1).astype(jnp.int32)
    acc = jnp.zeros((_BR, _S), jnp.float32)
    for k in range(_NB):
        acc = jnp.where(bucket == k, table_ref[k, 0], acc)
    return acc


def _add_bias_kernel(x_ref, table_ref, o_ref, bias_ref):
    h = pl.program_id(1)

    @pl.when(h == 0)
    def _():
        bias_ref[...] = _bias_block(pl.program_id(0), table_ref)

    o_ref[...] = x_ref[...] + bias_ref[...]


def _run(x, bias_table):
    bh, s, _ = x.shape
    grid = (s // _BR, bh)
    return pl.pallas_call(
        _add_bias_kernel,
        grid=grid,
        in_specs=[
            pl.BlockSpec((1, _BR, s), lambda r, hh: (hh, r, 0)),
            pl.BlockSpec((_NB, 1), lambda r, hh: (0, 0)),
        ],
        out_specs=pl.BlockSpec((1, _BR, s), lambda r, hh: (hh, r, 0)),
        out_shape=jax.ShapeDtypeStruct((bh, s, s), jnp.float32),
        scratch_shapes=[pltpu.VMEM((_BR, s), jnp.float32)],
        compiler_params=pltpu.CompilerParams(
            dimension_semantics=("parallel", "arbitrary")
        ),
    )(x, bias_table)


def kernel(attention_scores, bias_table):
    b, h, s, _ = attention_scores.shape
    x = attention_scores.reshape(b * h, s, s)
    nd = 1
    if nd > 1:
        devs = jax.devices()
        mesh = Mesh(np.array(devs[:nd]), ("d",))
        run = jax.shard_map(
            _run,
            mesh=mesh,
            in_specs=(P("d", None, None), P(None, None)),
            out_specs=P("d", None, None),
            check_vma=False,
        )
        out = run(x, bias_table)
    else:
        out = _run(x, bias_table)
    return out.reshape(b, h, s, s)


# speedup vs baseline: 4.1290x; 1.5042x over previous
import math

import jax
import jax.numpy as jnp
import numpy as np
from jax.experimental import pallas as pl
from jax.experimental.pallas import tpu as pltpu
from jax.sharding import Mesh, PartitionSpec as P

_NB = 32
_MD = 128
_BR = 512
_S = 2048


_WC = _S - 1
_WL = 4352


def _bias_bank():
    si = jax.lax.broadcasted_iota(jnp.int32, (8, _WL), 0)
    x = jax.lax.broadcasted_iota(jnp.int32, (8, _WL), 1)
    d = x - si - _WC
    rb = jnp.where(d > 0, _NB // 2, 0)
    a = jnp.abs(d)
    af = a.astype(jnp.float32)
    rp_if_large = _MD + jnp.log(af / _MD) / math.log(_MD / _NB) * (_NB - _MD)
    rp_if_large = jnp.minimum(rp_if_large, _MD - 1)
    large = rb.astype(jnp.float32) + rp_if_large
    small = (a + rb).astype(jnp.float32)
    out = jnp.where(a < _MD, small, large)
    return jnp.clip(out, 0, _NB - 1).astype(jnp.int32)


def _add_bias_kernel(x_ref, table_ref, o_ref, w_ref, bias_ref):
    r = pl.program_id(0)
    h = pl.program_id(1)

    @pl.when((h == 0) & (r == 0))
    def _():
        bucket = _bias_bank()
        acc = jnp.zeros((8, _WL), jnp.float32)
        for k in range(_NB):
            acc = jnp.where(bucket == k, table_ref[k, 0], acc)
        w_ref[...] = acc

    @pl.when(h == 0)
    def _():
        for g in range(_BR // 8):
            c = _WC - 8 * g
            rem = c % 128
            ba = (c - rem) - r * _BR
            chunk = w_ref[:, pl.ds(pl.multiple_of(ba, 128), _S + 128)]
            bias_ref[8 * g:8 * g + 8, :] = chunk[:, rem:rem + _S]

    o_ref[...] = x_ref[...] + bias_ref[...]


def _run(x, bias_table):
    bh, s, _ = x.shape
    grid = (s // _BR, bh)
    return pl.pallas_call(
        _add_bias_kernel,
        grid=grid,
        in_specs=[
            pl.BlockSpec((1, _BR, s), lambda r, hh: (hh, r, 0)),
            pl.BlockSpec((_NB, 1), lambda r, hh: (0, 0)),
        ],
        out_specs=pl.BlockSpec((1, _BR, s), lambda r, hh: (hh, r, 0)),
        out_shape=jax.ShapeDtypeStruct((bh, s, s), jnp.float32),
        scratch_shapes=[
            pltpu.VMEM((8, _WL), jnp.float32),
            pltpu.VMEM((_BR, s), jnp.float32),
        ],
        compiler_params=pltpu.CompilerParams(
            dimension_semantics=("parallel", "arbitrary")
        ),
    )(x, bias_table)


def kernel(attention_scores, bias_table):
    b, h, s, _ = attention_scores.shape
    x = attention_scores.reshape(b * h, s, s)
    nd = 1
    if nd > 1:
        devs = jax.devices()
        mesh = Mesh(np.array(devs[:nd]), ("d",))
        run = jax.shard_map(
            _run,
            mesh=mesh,
            in_specs=(P("d", None, None), P(None, None)),
            out_specs=P("d", None, None),
            check_vma=False,
        )
        out = run(x, bias_table)
    else:
        out = _run(x, bias_table)
    return out.reshape(b, h, s, s)
